# baseline (device time: 15156 ns/iter reference)
import jax
import jax.numpy as jnp
from jax import lax
from jax.experimental import pallas as pl
from jax.experimental.pallas import tpu as pltpu

N_DEV = 8
N_CHUNKS = 8


def kernel(x):
    m_per, n = x.shape
    assert m_per % N_CHUNKS == 0
    chunk = m_per // N_CHUNKS

    def body(x_ref, out_ref, acc_ref, comm_ref, send_sems, recv_sems):
        step = pl.program_id(0)
        my = lax.axis_index("i")

        partial = jnp.sum(x_ref[...], axis=0, keepdims=True)

        @pl.when(step == 0)
        def _():
            acc_ref[...] = partial

        @pl.when(step > 0)
        def _():
            acc_ref[...] = acc_ref[...] + partial

        @pl.when(step == N_CHUNKS - 1)
        def _():
            rdmas = []
            for j in range(1, N_DEV):
                dst = lax.rem(my + j, N_DEV)
                rdma = pltpu.make_async_remote_copy(
                    src_ref=acc_ref,
                    dst_ref=comm_ref.at[N_DEV - 1 - j],
                    send_sem=send_sems.at[j - 1],
                    recv_sem=recv_sems.at[N_DEV - 1 - j],
                    device_id=(dst,),
                    device_id_type=pl.DeviceIdType.MESH,
                )
                rdma.start()
                rdmas.append(rdma)
            for rdma in rdmas:
                rdma.wait()

            out_ref[...] = acc_ref[...] + jnp.sum(comm_ref[...], axis=0)

    return pl.pallas_call(
        body,
        grid=(N_CHUNKS,),
        out_shape=jax.ShapeDtypeStruct((1, n), x.dtype),
        in_specs=[pl.BlockSpec((chunk, n), lambda i: (i, 0))],
        out_specs=pl.BlockSpec((1, n), lambda i: (0, 0)),
        scratch_shapes=[
            pltpu.VMEM((1, n), x.dtype),
            pltpu.VMEM((N_DEV - 1, 1, n), x.dtype),
            pltpu.SemaphoreType.DMA((N_DEV - 1,)),
            pltpu.SemaphoreType.DMA((N_DEV - 1,)),
        ],
    )(x)


# device time: 10581 ns/iter; 1.4324x vs baseline; 1.4324x over previous
import jax
import jax.numpy as jnp
from jax import lax
from jax.experimental import pallas as pl
from jax.experimental.pallas import tpu as pltpu

N_DEV = 8


def kernel(x):
    m_per, n = x.shape

    def body(x_ref, out_ref, comm_ref, send_sems, recv_sems):
        my = lax.axis_index("i")

        barrier_sem = pltpu.get_barrier_semaphore()
        for j in range(1, N_DEV):
            pl.semaphore_signal(
                barrier_sem,
                inc=1,
                device_id=(lax.rem(my + j, N_DEV),),
                device_id_type=pl.DeviceIdType.MESH,
            )

        out_ref[...] = jnp.sum(x_ref[...], axis=0, keepdims=True)

        pl.semaphore_wait(barrier_sem, N_DEV - 1)

        rdmas = []
        for j in range(1, N_DEV):
            dst = lax.rem(my + j, N_DEV)
            rdma = pltpu.make_async_remote_copy(
                src_ref=out_ref,
                dst_ref=comm_ref.at[N_DEV - 1 - j],
                send_sem=send_sems.at[j - 1],
                recv_sem=recv_sems.at[N_DEV - 1 - j],
                device_id=(dst,),
                device_id_type=pl.DeviceIdType.MESH,
            )
            rdma.start()
            rdmas.append(rdma)
        for rdma in rdmas:
            rdma.wait()

        out_ref[...] = out_ref[...] + jnp.sum(comm_ref[...], axis=0)

    return pl.pallas_call(
        body,
        out_shape=jax.ShapeDtypeStruct((1, n), x.dtype),
        in_specs=[pl.BlockSpec(memory_space=pltpu.VMEM)],
        out_specs=pl.BlockSpec(memory_space=pltpu.VMEM),
        scratch_shapes=[
            pltpu.VMEM((N_DEV - 1, 1, n), x.dtype),
            pltpu.SemaphoreType.DMA((N_DEV - 1,)),
            pltpu.SemaphoreType.DMA((N_DEV - 1,)),
        ],
        compiler_params=pltpu.CompilerParams(collective_id=0),
    )(x)
